# baseline (speedup 1.0000x reference)
import functools

import jax
import jax.numpy as jnp
from jax import lax
from jax.experimental import pallas as pl
from jax.experimental.pallas import tpu as pltpu
from jax.experimental.pallas import tpu_sc as plsc

B = 16384
D_OUT = 96
FEATS = (
    (0, 32, 4, 2),
    (32, 16, 8, 3),
    (48, 16, 8, 3),
    (64, 16, 8, 3),
    (80, 16, 8, 3),
)
STREAMED = (0, 1, 4)
SMALL = (2, 3)

_info = plsc.get_sparse_core_info()
NC, NS = _info.num_cores, _info.num_subcores
NW = NC * NS
BPW = B // NW
CH = 32
NCH = BPW // CH
RING = 12

_mesh = plsc.VectorSubcoreMesh(core_axis_name="c", subcore_axis_name="s")


@functools.partial(
    pl.kernel,
    mesh=_mesh,
    out_type=jax.ShapeDtypeStruct((D_OUT, B), jnp.float32),
    scratch_types=[
        pltpu.VMEM((BPW,), jnp.int32),
        pltpu.VMEM((BPW,), jnp.int32),
        pltpu.VMEM((BPW,), jnp.int32),
        pltpu.VMEM((BPW,), jnp.int32),
        pltpu.VMEM((BPW,), jnp.int32),
        pltpu.VMEM((BPW,), jnp.int32),
        pltpu.VMEM((BPW,), jnp.int32),
        pltpu.VMEM((BPW,), jnp.int32),
        pltpu.VMEM((RING, CH, 128), jnp.float32),
        pltpu.VMEM((32, BPW), jnp.float32),
        pltpu.VMEM((16, BPW), jnp.float32),
        pltpu.VMEM((16, BPW), jnp.float32),
        pltpu.VMEM((16, BPW), jnp.float32),
        pltpu.VMEM((16, BPW), jnp.float32),
        pltpu.VMEM((3, 128), jnp.float32),
        pltpu.VMEM((1, 128), jnp.float32),
    ] + [pltpu.SemaphoreType.DMA] * (RING + 1),
    compiler_params=pltpu.CompilerParams(needs_layout_passes=False),
)
def _gather_concat(acc_i, area_i, hour_i, wk_i, geo_i,
                   acc_t, area_t, hour_t, wk_t, geo_t,
                   out,
                   ix0, ix1, ix2, ix3, ix4, dv0, dv1, dv4,
                   gbuf, e0, e1, e2, e3, e4, tabh, tabw,
                   *sems):
    wsem = sems[RING]
    wid = lax.axis_index("s") * NC + lax.axis_index("c")
    base = wid * BPW

    idx_refs = (acc_i, area_i, hour_i, wk_i, geo_i)
    tabs = (acc_t, area_t, hour_t, wk_t, geo_t)
    ivs = (ix0, ix1, ix2, ix3, ix4)
    dvs = {0: dv0, 1: dv1, 4: dv4}
    ebufs = (e0, e1, e2, e3, e4)

    for f in range(5):
        pltpu.sync_copy(idx_refs[f].at[pl.ds(base, BPW)], ivs[f])
    pltpu.sync_copy(hour_t, tabh)
    pltpu.sync_copy(wk_t, tabw)

    def divbody(g, _):
        for f in STREAMED:
            v = ivs[f][pl.ds(g * 16, 16)]
            dvs[f][pl.ds(g * 16, 16)] = lax.shift_right_logical(
                v, FEATS[f][3])
        return 0

    lax.fori_loop(0, BPW // 16, divbody, 0, unroll=2)

    iota16 = lax.iota(jnp.int32, 16)
    pairs = [(f, j) for f in STREAMED for j in range(NCH)]

    def fire(p):
        f, j = pairs[p]
        return pltpu.async_copy(
            tabs[f].at[dvs[f].at[pl.ds(j * CH, CH)]],
            gbuf.at[p % RING], sems[p % RING])

    desc = [fire(p) for p in range(RING)]

    def smallbody(g, _):
        for f in SMALL:
            _, emb, k, sh = FEATS[f]
            ids = ivs[f][pl.ds(g * 16, 16)]
            row = lax.shift_right_logical(ids, sh)
            col0 = jnp.bitwise_and(ids, k - 1) * emb
            tab = tabh if f == 2 else tabw
            eb = ebufs[f]
            for d in range(emb):
                vals = plsc.load_gather(tab, [row, col0 + d])
                eb[d, pl.ds(g * 16, 16)] = vals
        return 0

    lax.fori_loop(0, BPW // 16, smallbody, 0)

    for p in range(len(pairs)):
        desc[p % RING].wait()
        f, j = pairs[p]
        _, emb, k, _ = FEATS[f]
        gb = gbuf.at[p % RING]
        eb = ebufs[f]
        iv = ivs[f]

        def extbody(g, _, j=j, emb=emb, k=k, gb=gb, eb=eb, iv=iv):
            ids = iv[pl.ds(j * CH + g * 16, 16)]
            col0 = jnp.bitwise_and(ids, k - 1) * emb
            rvec = iota16 + g * 16
            for d in range(emb):
                vals = plsc.load_gather(gb, [rvec, col0 + d])
                eb[d, pl.ds(j * CH + g * 16, 16)] = vals
            return 0

        lax.fori_loop(0, CH // 16, extbody, 0)
        if p + RING < len(pairs):
            desc[p % RING] = fire(p + RING)

    wcopies = []
    for f in range(5):
        col, emb = FEATS[f][0], FEATS[f][1]
        wcopies.append(
            pltpu.async_copy(
                ebufs[f], out.at[pl.ds(col, emb), pl.ds(base, BPW)], wsem))
    for w in wcopies:
        w.wait()


def _pack(table, k):
    v = table.shape[0]
    pad = (-v) % k
    if pad:
        table = jnp.pad(table, ((0, pad), (0, 0)))
    return jnp.reshape(table, (-1, 128))


def kernel(account_id, delivery_area_id, order_hour, order_weekday, geohash6,
           account_table, area_table, hour_table, weekday_table, geohash_table):
    out_t = _gather_concat(
        account_id.astype(jnp.int32), delivery_area_id.astype(jnp.int32),
        order_hour.astype(jnp.int32), order_weekday.astype(jnp.int32),
        geohash6.astype(jnp.int32),
        _pack(account_table, 4), _pack(area_table, 8), _pack(hour_table, 8),
        _pack(weekday_table, 8), _pack(geohash_table, 8))
    return out_t.T


# speedup vs baseline: 1.6248x; 1.6248x over previous
import functools

import jax
import jax.numpy as jnp
from jax import lax
from jax.experimental import pallas as pl
from jax.experimental.pallas import tpu as pltpu
from jax.experimental.pallas import tpu_sc as plsc

B = 16384
D_OUT = 96
COLS = (0, 32, 48, 64, 80)
DIMS = (32, 16, 16, 16, 16)

_info = plsc.get_sparse_core_info()
NC, NS = _info.num_cores, _info.num_subcores
NW = NC * NS
BPW = B // NW
CH = 64
NCH = BPW // CH

_mesh = plsc.VectorSubcoreMesh(core_axis_name="c", subcore_axis_name="s")


@functools.partial(
    pl.kernel,
    mesh=_mesh,
    out_type=jax.ShapeDtypeStruct((B, D_OUT), jnp.float32),
    scratch_types=[
        pltpu.VMEM((BPW,), jnp.int32),
        pltpu.VMEM((BPW,), jnp.int32),
        pltpu.VMEM((BPW,), jnp.int32),
        pltpu.VMEM((BPW,), jnp.int32),
        pltpu.VMEM((BPW,), jnp.int32),
        pltpu.VMEM((BPW, 32), jnp.float32),
        pltpu.VMEM((BPW, 16), jnp.float32),
        pltpu.VMEM((BPW, 16), jnp.float32),
        pltpu.VMEM((BPW, 16), jnp.float32),
        pltpu.VMEM((BPW, 16), jnp.float32),
        pltpu.VMEM((24, 16), jnp.float32),
        pltpu.VMEM((7, 16), jnp.float32),
        pltpu.SemaphoreType.DMA,
        pltpu.SemaphoreType.DMA,
    ],
    compiler_params=pltpu.CompilerParams(
        use_tc_tiling_on_sc=False, needs_layout_passes=False),
)
def _gather_concat(acc_i, area_i, hour_i, wk_i, geo_i,
                   acc_t, area_t, hour_t, wk_t, geo_t,
                   out,
                   ix0, ix1, ix2, ix3, ix4,
                   acc_v, area_v, hour_v, wk_v, geo_v,
                   tabh, tabw, sem, wsem):
    wid = lax.axis_index("s") * NC + lax.axis_index("c")
    base = wid * BPW

    idx_refs = (acc_i, area_i, hour_i, wk_i, geo_i)
    ivs = (ix0, ix1, ix2, ix3, ix4)
    for f in range(5):
        pltpu.sync_copy(idx_refs[f].at[pl.ds(base, BPW)], ivs[f])
    pltpu.sync_copy(hour_t, tabh)
    pltpu.sync_copy(wk_t, tabw)

    copies = []
    for iv, tab, buf in ((ix0, acc_t, acc_v), (ix1, area_t, area_v),
                         (ix4, geo_t, geo_v)):
        for j in range(NCH):
            copies.append(
                pltpu.async_copy(tab.at[iv.at[pl.ds(j * CH, CH)]],
                                 buf.at[pl.ds(j * CH, CH)], sem))

    iota16 = lax.iota(jnp.int32, 16)

    def smallbody(g, _):
        rvec = iota16 + g * 16
        idsh = ix2[pl.ds(g * 16, 16)]
        idsw = ix3[pl.ds(g * 16, 16)]
        for d in range(16):
            dv = jnp.full((16,), d, jnp.int32)
            vh = plsc.load_gather(tabh, [idsh, dv])
            plsc.store_scatter(hour_v, [rvec, dv], vh)
            vw = plsc.load_gather(tabw, [idsw, dv])
            plsc.store_scatter(wk_v, [rvec, dv], vw)
        return 0

    lax.fori_loop(0, BPW // 16, smallbody, 0)

    for c in copies:
        c.wait()

    bufs = (acc_v, area_v, hour_v, wk_v, geo_v)
    wcopies = []
    for f in range(5):
        wcopies.append(
            pltpu.async_copy(
                bufs[f], out.at[pl.ds(base, BPW), pl.ds(COLS[f], DIMS[f])],
                wsem))
    for w in wcopies:
        w.wait()


def kernel(account_id, delivery_area_id, order_hour, order_weekday, geohash6,
           account_table, area_table, hour_table, weekday_table, geohash_table):
    return _gather_concat(
        account_id.astype(jnp.int32), delivery_area_id.astype(jnp.int32),
        order_hour.astype(jnp.int32), order_weekday.astype(jnp.int32),
        geohash6.astype(jnp.int32),
        account_table, area_table, hour_table, weekday_table, geohash_table)
